# baseline (speedup 1.0000x reference)
import numpy as np
import jax
import jax.numpy as jnp
from jax.experimental import pallas as pl

B = 4096
BN = 256
F32 = jnp.float32
DT = jnp.bfloat16


def _sel(njj, nx, width):
    s = np.zeros((njj, 2, nx, 5), np.float32)
    for jj in range(njj):
        for p in range(2):
            for dj in range(5):
                jx = 2 * jj + p + dj
                if jx < nx:
                    s[jj, p, jx, dj] = 1.0
    return s


_SEL1 = _sel(14, 32, 5)
_SEL2 = _sel(5, 14, 5)


def _mega_kernel(xt_ref, m1_ref, b1p_ref, m2_ref, b2p_ref,
                 fc1_ref, fc1b_ref, fc2_ref, fc2b_ref, gw_ref,
                 w1c_ref, b1c_ref, w2c_ref, b2c_ref, fc3_ref, fc3b_ref,
                 out_ref):
    xv = xt_ref[...].astype(DT)
    m1 = m1_ref[...]
    m2 = m2_ref[...]

    h1rows = []
    for jj in range(14):
        ys = []
        for p in range(2):
            i = 2 * jj + p
            s = jnp.concatenate(
                [xv[c * 1024 + (i + di) * 32:c * 1024 + (i + di) * 32 + 32]
                 for c in range(3) for di in range(5)], axis=0)
            ys.append(jnp.dot(m1, s, preferred_element_type=F32))
        y = jnp.maximum(ys[0], ys[1])
        h1 = jnp.maximum(y[0:96], y[96:192])
        h1 = jnp.maximum(h1 + b1p_ref[...], 0.0).astype(DT)
        h1rows.append(h1)

    h2rows = []
    for jj in range(5):
        ys = []
        for p in range(2):
            i = 2 * jj + p
            s = jnp.concatenate([h1rows[i + di] for di in range(5)], axis=0)
            ys.append(jnp.dot(m2, s, preferred_element_type=F32))
        y = jnp.maximum(ys[0], ys[1])
        h2 = jnp.maximum(y[0:80], y[80:160])
        h2 = jnp.maximum(h2 + b2p_ref[...], 0.0).astype(DT)
        h2rows.append(h2)

    hp = jnp.concatenate(h2rows, axis=0)
    z1 = jnp.maximum(jnp.dot(fc1_ref[...], hp, preferred_element_type=F32)
                     + fc1b_ref[...], 0.0).astype(DT)
    h = jnp.maximum(jnp.dot(fc2_ref[...], z1, preferred_element_type=F32)
                    + fc2b_ref[...], 0.0).astype(DT)

    logits = jnp.dot(gw_ref[...], h, preferred_element_type=F32)
    mx = jnp.max(logits, axis=0, keepdims=True)
    e = jnp.exp(logits - mx)
    g = e / jnp.sum(e, axis=0, keepdims=True)

    eh = jnp.maximum(jnp.dot(w1c_ref[...], h, preferred_element_type=F32)
                     + b1c_ref[...], 0.0)
    rows = jax.lax.broadcasted_iota(jnp.int32, (4096, BN), 0)
    gfull = jnp.where(rows < 2048, g[0:1, :], g[1:2, :])
    ehs = (eh * gfull).astype(DT)
    moe = (jnp.dot(w2c_ref[...], ehs, preferred_element_type=F32)
           + jnp.dot(b2c_ref[...], g, preferred_element_type=F32))

    res = (jnp.dot(fc3_ref[...], moe, preferred_element_type=F32)
           + fc3b_ref[...])
    out_ref[...] = jnp.transpose(res)


def kernel(x, conv1_w, conv1_b, conv2_w, conv2_b, fc1_w, fc1_b, fc2_w, fc2_b,
           gate_w, exp_w1, exp_b1, exp_w2, exp_b2, fc3_w, fc3_b):
    xt = x.transpose(1, 2, 3, 0).reshape(3072, B)

    m1 = jnp.einsum('ocde,jpxe->pojcdx', conv1_w, _SEL1)
    m1 = m1.reshape(2, 84, 480)
    m1 = jnp.pad(m1, ((0, 0), (0, 12), (0, 0))).reshape(192, 480).astype(DT)
    m2 = jnp.einsum('ocde,jpxe->pojdcx', conv2_w, _SEL2)
    m2 = m2.reshape(2, 80, 5, 84)
    m2 = jnp.pad(m2, ((0, 0), (0, 0), (0, 0), (0, 12))).reshape(160, 480).astype(DT)

    b1p = jnp.pad(jnp.repeat(conv1_b, 14), (0, 12))[:, None]
    b2p = jnp.repeat(conv2_b, 5)[:, None]
    fc1p = fc1_w.reshape(120, 16, 5, 5).transpose(0, 2, 1, 3).reshape(120, 400).astype(DT)
    w1c = exp_w1.transpose(0, 2, 1).reshape(4096, 84).astype(DT)
    b1c = exp_b1.reshape(4096, 1)
    w2c = exp_w2.transpose(2, 0, 1).reshape(84, 4096).astype(DT)
    b2c = exp_b2.T

    rep = lambda *_: tuple(0 for _ in range(2))
    full2 = lambda a: pl.BlockSpec(a.shape, lambda i: (0, 0))

    out = pl.pallas_call(
        _mega_kernel,
        grid=(B // BN,),
        in_specs=[
            pl.BlockSpec((3072, BN), lambda i: (0, i)),
            full2(m1), full2(b1p), full2(m2), full2(b2p),
            full2(fc1p), full2(fc1_b[:, None]),
            full2(fc2_w.astype(DT)), full2(fc2_b[:, None]),
            full2(gate_w.T.astype(DT)),
            full2(w1c), full2(b1c), full2(w2c), full2(b2c),
            full2(fc3_w), full2(fc3_b[:, None]),
        ],
        out_specs=pl.BlockSpec((BN, 10), lambda i: (i, 0)),
        out_shape=jax.ShapeDtypeStruct((B, 10), F32),
    )(xt, m1, b1p, m2, b2p, fc1p, fc1_b[:, None], fc2_w.astype(DT), fc2_b[:, None],
      gate_w.T.astype(DT), w1c, b1c, w2c, b2c, fc3_w, fc3_b[:, None])
    return out


# speedup vs baseline: 1.1399x; 1.1399x over previous
import numpy as np
import jax
import jax.numpy as jnp
from jax.experimental import pallas as pl

B = 4096
BN = 256
F32 = jnp.float32
DT = jnp.bfloat16


def _sel(njj, nx, width):
    s = np.zeros((njj, 2, nx, 5), np.float32)
    for jj in range(njj):
        for p in range(2):
            for dj in range(5):
                jx = 2 * jj + p + dj
                if jx < nx:
                    s[jj, p, jx, dj] = 1.0
    return s


_SEL1 = _sel(14, 32, 5)
_SEL2 = _sel(5, 14, 5)


def _mega_kernel(xt_ref, m1_ref, b1p_ref, m2_ref, b2p_ref,
                 fc1_ref, fc1b_ref, fc2_ref, fc2b_ref, gw_ref,
                 w1c_ref, b1c_ref, w2c_ref, b2c_ref, fc3_ref, fc3b_ref,
                 out_ref):
    xv = xt_ref[...].astype(DT)
    m1 = m1_ref[...]
    m2 = m2_ref[...]

    h1rows = []
    for jj in range(14):
        ys = []
        for p in range(2):
            i = 2 * jj + p
            s = jnp.concatenate(
                [xv[c * 1024 + (i + di) * 32:c * 1024 + (i + di) * 32 + 32]
                 for c in range(3) for di in range(5)], axis=0)
            ys.append(jnp.dot(m1, s, preferred_element_type=F32))
        y = jnp.maximum(ys[0], ys[1])
        h1 = jnp.maximum(y[0:96], y[96:192])
        h1 = jnp.maximum(h1 + b1p_ref[...], 0.0).astype(DT)
        h1rows.append(h1)

    h2rows = []
    for jj in range(5):
        ys = []
        for p in range(2):
            i = 2 * jj + p
            s = jnp.concatenate([h1rows[i + di] for di in range(5)], axis=0)
            ys.append(jnp.dot(m2, s, preferred_element_type=F32))
        y = jnp.maximum(ys[0], ys[1])
        h2 = jnp.maximum(y[0:80], y[80:160])
        h2 = jnp.maximum(h2 + b2p_ref[...], 0.0).astype(DT)
        h2rows.append(h2)

    hp = jnp.concatenate(h2rows, axis=0)
    z1 = jnp.maximum(jnp.dot(fc1_ref[...], hp, preferred_element_type=F32)
                     + fc1b_ref[...], 0.0).astype(DT)
    h = jnp.maximum(jnp.dot(fc2_ref[...], z1, preferred_element_type=F32)
                    + fc2b_ref[...], 0.0).astype(DT)

    logits = jnp.dot(gw_ref[...], h, preferred_element_type=F32)
    mx = jnp.max(logits, axis=0, keepdims=True)
    e = jnp.exp(logits - mx)
    g = e / jnp.sum(e, axis=0, keepdims=True)

    eh = jnp.maximum(jnp.dot(w1c_ref[...], h, preferred_element_type=F32)
                     + b1c_ref[...], 0.0)
    rows = jax.lax.broadcasted_iota(jnp.int32, (4096, BN), 0)
    gfull = jnp.where(rows < 2048, g[0:1, :], g[1:2, :])
    ehs = (eh * gfull).astype(DT)
    moe = (jnp.dot(w2c_ref[...], ehs, preferred_element_type=F32)
           + jnp.dot(b2c_ref[...], g, preferred_element_type=F32))

    out_ref[...] = (jnp.dot(fc3_ref[...], moe, preferred_element_type=F32)
                    + fc3b_ref[...])


def kernel(x, conv1_w, conv1_b, conv2_w, conv2_b, fc1_w, fc1_b, fc2_w, fc2_b,
           gate_w, exp_w1, exp_b1, exp_w2, exp_b2, fc3_w, fc3_b):
    xt = x.transpose(1, 2, 3, 0).reshape(3072, B)

    m1 = jnp.einsum('ocde,jpxe->pojcdx', conv1_w, _SEL1)
    m1 = m1.reshape(2, 84, 480)
    m1 = jnp.pad(m1, ((0, 0), (0, 12), (0, 0))).reshape(192, 480).astype(DT)
    m2 = jnp.einsum('ocde,jpxe->pojdcx', conv2_w, _SEL2)
    m2 = m2.reshape(2, 80, 5, 84)
    m2 = jnp.pad(m2, ((0, 0), (0, 0), (0, 0), (0, 12))).reshape(160, 480).astype(DT)

    b1p = jnp.pad(jnp.repeat(conv1_b, 14), (0, 12))[:, None]
    b2p = jnp.repeat(conv2_b, 5)[:, None]
    fc1p = fc1_w.reshape(120, 16, 5, 5).transpose(0, 2, 1, 3).reshape(120, 400).astype(DT)
    w1c = exp_w1.transpose(0, 2, 1).reshape(4096, 84).astype(DT)
    b1c = exp_b1.reshape(4096, 1)
    w2c = exp_w2.transpose(2, 0, 1).reshape(84, 4096).astype(DT)
    b2c = exp_b2.T

    rep = lambda *_: tuple(0 for _ in range(2))
    full2 = lambda a: pl.BlockSpec(a.shape, lambda i: (0, 0))

    out = pl.pallas_call(
        _mega_kernel,
        grid=(B // BN,),
        in_specs=[
            pl.BlockSpec((3072, BN), lambda i: (0, i)),
            full2(m1), full2(b1p), full2(m2), full2(b2p),
            full2(fc1p), full2(fc1_b[:, None]),
            full2(fc2_w.astype(DT)), full2(fc2_b[:, None]),
            full2(gate_w.T.astype(DT)),
            full2(w1c), full2(b1c), full2(w2c), full2(b2c),
            full2(fc3_w), full2(fc3_b[:, None]),
        ],
        out_specs=pl.BlockSpec((10, BN), lambda i: (0, i)),
        out_shape=jax.ShapeDtypeStruct((10, B), F32),
    )(xt, m1, b1p, m2, b2p, fc1p, fc1_b[:, None], fc2_w.astype(DT), fc2_b[:, None],
      gate_w.T.astype(DT), w1c, b1c, w2c, b2c, fc3_w, fc3_b[:, None])
    return out.T
